# Initial kernel scaffold; baseline (speedup 1.0000x reference)
import jax
import jax.numpy as jnp
from jax.experimental import pallas as pl


def kernel(x_author, x_paper, edge_index_writes, edge_index_rev_writes, p1w_author, p1b_author, p1w_paper, p1b_paper, a1s_writes, a1d_writes, a1s_rev, a1d_rev, k1w, k1b, q1, p2w_author, p2b_author, p2w_paper, p2b_paper, a2s_writes, a2d_writes, a2s_rev, a2d_rev, k2w, k2b, q2):
    raise NotImplementedError("write your pallas kernel here")


# speedup vs baseline: 76.1340x; 76.1340x over previous
import functools

import jax
import jax.numpy as jnp
from jax import lax
from jax.experimental import pallas as pl
from jax.experimental.pallas import tpu as pltpu
from jax.experimental.pallas import tpu_sc as plsc

N = 10000
E = 320000
CF = 128
NH = 8
HD = 16

NC = 2
NS = 16
NW = NC * NS
CHUNK = 128
K = 80
EPT = K * CHUNK
EP = EPT * NW
K_LAST = (E - (NW - 1) * EPT) // CHUNK
ROWS_PT = 640
RQ = 80

_f32 = jnp.float32


def _proj_body(x_ref, w_ref, b_ref, wl_ref, h_ref, l_ref):
    h = jnp.dot(x_ref[...], w_ref[...], preferred_element_type=_f32)
    h = h + b_ref[...]
    h_ref[...] = h
    l_ref[...] = jnp.dot(h, wl_ref[...], preferred_element_type=_f32)


def _tc_proj(x, w, b, wl):
    blk = 1000
    return pl.pallas_call(
        _proj_body,
        grid=(N // blk,),
        in_specs=[
            pl.BlockSpec((blk, CF), lambda i: (i, 0)),
            pl.BlockSpec((CF, CF), lambda i: (0, 0)),
            pl.BlockSpec((1, CF), lambda i: (0, 0)),
            pl.BlockSpec((CF, 2 * NH), lambda i: (0, 0)),
        ],
        out_specs=[
            pl.BlockSpec((blk, CF), lambda i: (i, 0)),
            pl.BlockSpec((blk, 2 * NH), lambda i: (i, 0)),
        ],
        out_shape=[
            jax.ShapeDtypeStruct((N, CF), _f32),
            jax.ShapeDtypeStruct((N, 2 * NH), _f32),
        ],
    )(x, w.astype(_f32), b.reshape(1, CF).astype(_f32), wl)


def _att_block(att):
    eye = jnp.eye(NH, dtype=_f32)
    return (eye[:, None, :] * att[:, :, None]).reshape(CF, NH)


def _make_sc_edge(hoff):
    mesh = plsc.VectorSubcoreMesh(core_axis_name="c", subcore_axis_name="s")

    @functools.partial(
        pl.kernel,
        out_type=[
            jax.ShapeDtypeStruct((NC, N, CF), _f32),
            jax.ShapeDtypeStruct((NC, N, 2 * NH), _f32),
        ],
        mesh=mesh,
        compiler_params=pltpu.CompilerParams(use_tc_tiling_on_sc=False),
        scratch_types=[
            pltpu.VMEM((K, CHUNK), jnp.int32),
            pltpu.VMEM((K, CHUNK), jnp.int32),
            pltpu.VMEM((CHUNK, 2 * NH), _f32),
            pltpu.VMEM((CHUNK, 2 * NH), _f32),
            pltpu.VMEM((CHUNK, CF), _f32),
            pltpu.VMEM_SHARED((N, CF), _f32),
            pltpu.VMEM_SHARED((N, 2 * NH), _f32),
            pltpu.SemaphoreType.DMA,
            pltpu.SemaphoreType.DMA,
        ],
    )
    def sc_edge(h_hbm, slog_hbm, dlog_hbm, src_hbm, dst_hbm,
                un_hbm, den_hbm,
                idx_s, idx_d, asb, adb, hb, out_sp, den_sp, sem, sem2):
        c = lax.axis_index("c")
        s = lax.axis_index("s")
        wid = s * NC + c

        z16 = jnp.zeros((HD,), _f32)

        def _zrow(i, _):
            for kk in range(CF // HD):
                hb[i, pl.ds(HD * kk, HD)] = z16
            asb[i, :] = z16
            return 0

        lax.fori_loop(0, CHUNK, _zrow, 0)
        rbase = s * ROWS_PT
        nq = jnp.where(s == NS - 1, (N - (NS - 1) * ROWS_PT) // RQ,
                       ROWS_PT // RQ)

        def _zq(kk, _):
            pltpu.sync_copy(hb.at[pl.ds(0, RQ)],
                            out_sp.at[pl.ds(rbase + RQ * kk, RQ)])
            pltpu.sync_copy(asb.at[pl.ds(0, RQ)],
                            den_sp.at[pl.ds(rbase + RQ * kk, RQ)])
            return 0

        lax.fori_loop(0, nq, _zq, 0)
        plsc.subcore_barrier()

        pltpu.sync_copy(src_hbm.at[pl.ds(wid * K, K)], idx_s)
        pltpu.sync_copy(dst_hbm.at[pl.ds(wid * K, K)], idx_d)

        def _chunk(j, _):
            isrc = idx_s.at[j]
            idst = idx_d.at[j]
            cp1 = pltpu.async_copy(slog_hbm.at[isrc], asb, sem)
            cp2 = pltpu.async_copy(dlog_hbm.at[idst], adb, sem2)
            cp1.wait()
            cp2.wait()

            def _erow(i, _):
                t = asb[i, :] + adb[i, :]
                t = jnp.where(t > 0, t, 0.2 * t)
                asb[i, :] = jnp.exp(t)
                return 0

            lax.fori_loop(0, CHUNK, _erow, 0)
            pltpu.sync_copy(asb, den_sp.at[idst], add=True)

            pltpu.async_copy(h_hbm.at[isrc], hb, sem).wait()

            def _mrow(i, _):
                ex_row = asb[i, :]
                for hh in range(NH):
                    sc = ex_row[hoff + hh]
                    hb[i, pl.ds(HD * hh, HD)] = hb[i, pl.ds(HD * hh, HD)] * sc
                return 0

            lax.fori_loop(0, CHUNK, _mrow, 0)
            pltpu.sync_copy(hb, out_sp.at[idst], add=True)
            return 0

        nch = jnp.where(wid == NW - 1, K_LAST, K)
        lax.fori_loop(0, nch, _chunk, 0)
        plsc.subcore_barrier()

        def _fq(kk, _):
            r0 = rbase + RQ * kk
            pltpu.sync_copy(out_sp.at[pl.ds(r0, RQ)],
                            un_hbm.at[c, pl.ds(r0, RQ)])
            pltpu.sync_copy(den_sp.at[pl.ds(r0, RQ)],
                            den_hbm.at[c, pl.ds(r0, RQ)])
            return 0

        lax.fori_loop(0, nq, _fq, 0)

    return sc_edge


_sc_edge_w = _make_sc_edge(0)
_sc_edge_r = _make_sc_edge(NH)


def _make_norm_body(off):
    def _norm_body(un_ref, den_ref, o_ref):
        u = un_ref[0] + un_ref[1]
        d = den_ref[0] + den_ref[1]
        dh = d[:, off:off + NH]
        dr = jnp.broadcast_to(dh[:, :, None], dh.shape + (HD,))
        dr = dr.reshape(u.shape)
        o_ref[...] = jnp.maximum(u / (dr + 1e-16), 0.0)
    return _norm_body


def _tc_norm(un, den, off):
    blk = 1000
    return pl.pallas_call(
        _make_norm_body(off),
        grid=(N // blk,),
        in_specs=[
            pl.BlockSpec((NC, blk, CF), lambda i: (0, i, 0)),
            pl.BlockSpec((NC, blk, 2 * NH), lambda i: (0, i, 0)),
        ],
        out_specs=pl.BlockSpec((blk, CF), lambda i: (i, 0)),
        out_shape=jax.ShapeDtypeStruct((N, CF), _f32),
    )(un, den)


def _pad_edges(ei):
    src = jnp.pad(ei[0], (0, EP - E)).reshape(EP // CHUNK, CHUNK)
    dst = jnp.pad(ei[1], (0, EP - E)).reshape(EP // CHUNK, CHUNK)
    return src, dst


def _layer(x_a, x_p, ew, er, pw_a, pb_a, pw_p, pb_p,
           as_w, ad_w, as_r, ad_r):
    wl_a = jnp.concatenate([_att_block(as_w), _att_block(ad_r)], axis=1)
    wl_p = jnp.concatenate([_att_block(ad_w), _att_block(as_r)], axis=1)
    h_a, la = _tc_proj(x_a, pw_a, pb_a, wl_a)
    h_p, lp = _tc_proj(x_p, pw_p, pb_p, wl_p)
    sw, dw = ew
    sr, dr = er
    un_p, den_p = _sc_edge_w(h_a, la, lp, sw, dw)
    un_a, den_a = _sc_edge_r(h_p, lp, la, sr, dr)
    out_a = _tc_norm(un_a, den_a, NH)
    out_p = _tc_norm(un_p, den_p, 0)
    return out_a, out_p


def kernel(x_author, x_paper, edge_index_writes, edge_index_rev_writes,
           p1w_author, p1b_author, p1w_paper, p1b_paper,
           a1s_writes, a1d_writes, a1s_rev, a1d_rev, k1w, k1b, q1,
           p2w_author, p2b_author, p2w_paper, p2b_paper,
           a2s_writes, a2d_writes, a2s_rev, a2d_rev, k2w, k2b, q2):
    ew = _pad_edges(edge_index_writes)
    er = _pad_edges(edge_index_rev_writes)
    a1, p1 = _layer(x_author, x_paper, ew, er,
                    p1w_author, p1b_author, p1w_paper, p1b_paper,
                    a1s_writes, a1d_writes, a1s_rev, a1d_rev)
    a2, p2 = _layer(a1, p1, ew, er,
                    p2w_author, p2b_author, p2w_paper, p2b_paper,
                    a2s_writes, a2d_writes, a2s_rev, a2d_rev)
    return jnp.stack([a2, p2])


# speedup vs baseline: 105.1437x; 1.3810x over previous
import functools

import jax
import jax.numpy as jnp
from jax import lax
from jax.experimental import pallas as pl
from jax.experimental.pallas import tpu as pltpu
from jax.experimental.pallas import tpu_sc as plsc

N = 10000
E = 320000
CF = 128
NH = 8
HD = 16

NC = 2
NS = 16
NW = NC * NS
HC = 56
NST = 180
EPT = HC * NST
EP = EPT * NW
N1 = N + 8
ROWS_PT = 640
RQ = 40
RING = 8

_f32 = jnp.float32


def _proj_body(x_ref, w_ref, b_ref, wl_ref, h_ref, l_ref):
    h = jnp.dot(x_ref[...], w_ref[...], preferred_element_type=_f32)
    h = h + b_ref[...]
    h_ref[...] = h
    l_ref[...] = jnp.dot(h, wl_ref[...], preferred_element_type=_f32)


def _tc_proj(x, w, b, wl):
    blk = 1000
    return pl.pallas_call(
        _proj_body,
        grid=(N // blk,),
        in_specs=[
            pl.BlockSpec((blk, CF), lambda i: (i, 0)),
            pl.BlockSpec((CF, CF), lambda i: (0, 0)),
            pl.BlockSpec((1, CF), lambda i: (0, 0)),
            pl.BlockSpec((CF, 2 * NH), lambda i: (0, 0)),
        ],
        out_specs=[
            pl.BlockSpec((blk, CF), lambda i: (i, 0)),
            pl.BlockSpec((blk, 2 * NH), lambda i: (i, 0)),
        ],
        out_shape=[
            jax.ShapeDtypeStruct((N, CF), _f32),
            jax.ShapeDtypeStruct((N, 2 * NH), _f32),
        ],
    )(x, w.astype(_f32), b.reshape(1, CF).astype(_f32), wl)


def _att_block(att):
    eye = jnp.eye(NH, dtype=_f32)
    return (eye[:, None, :] * att[:, :, None]).reshape(CF, NH)


def _make_sc_edge(hoff):
    mesh = plsc.VectorSubcoreMesh(core_axis_name="c", subcore_axis_name="s")

    set_types = []
    for _ in range(4):
        set_types += [
            pltpu.VMEM((HC, 2 * NH), _f32),
            pltpu.VMEM((HC, 2 * NH), _f32),
            pltpu.VMEM((HC, CF), _f32),
            pltpu.SemaphoreType.DMA,
            pltpu.SemaphoreType.DMA,
        ]

    @functools.partial(
        pl.kernel,
        out_type=[
            jax.ShapeDtypeStruct((NC, N, CF), _f32),
            jax.ShapeDtypeStruct((NC, N, 2 * NH), _f32),
        ],
        mesh=mesh,
        compiler_params=pltpu.CompilerParams(use_tc_tiling_on_sc=False),
        scratch_types=[
            pltpu.VMEM((RING, HC), jnp.int32),
            pltpu.VMEM((RING, HC), jnp.int32),
            pltpu.VMEM_SHARED((N1, CF), _f32),
            pltpu.VMEM_SHARED((N1, 2 * NH), _f32),
            pltpu.SemaphoreType.DMA,
            pltpu.SemaphoreType.DMA,
        ] + set_types,
    )
    def sc_edge(h_hbm, slog_hbm, dlog_hbm, src_hbm, dst_hbm,
                un_hbm, den_hbm,
                rs, rd, out_sp, den_sp, ie, io, *bufsets):
        sets = [bufsets[5 * b:5 * b + 5] for b in range(4)]
        isems = (ie, io)
        c = lax.axis_index("c")
        s = lax.axis_index("s")
        wid = s * NC + c
        ebase = wid * EPT

        la0, ld0, hf0 = sets[0][0], sets[0][1], sets[0][2]

        z16 = jnp.zeros((HD,), _f32)

        def _zrow(i, _):
            for kk in range(CF // HD):
                hf0[i, pl.ds(HD * kk, HD)] = z16
            la0[i, :] = z16
            return 0

        lax.fori_loop(0, HC, _zrow, 0)
        rbase = s * ROWS_PT
        nq = jnp.where(s == NS - 1, (N - (NS - 1) * ROWS_PT) // RQ,
                       ROWS_PT // RQ)

        def _zq(kk, _):
            pltpu.sync_copy(hf0.at[pl.ds(0, RQ)],
                            out_sp.at[pl.ds(rbase + RQ * kk, RQ)])
            pltpu.sync_copy(la0.at[pl.ds(0, RQ)],
                            den_sp.at[pl.ds(rbase + RQ * kk, RQ)])
            return 0

        lax.fori_loop(0, nq, _zq, 0)
        plsc.subcore_barrier()

        def _slot(m):
            return jax.lax.rem(m, RING)

        def _issue_idx(m, par):
            sl = _slot(m)
            pltpu.async_copy(src_hbm.at[pl.ds(ebase + m * HC, HC)],
                             rs.at[sl], isems[par])
            pltpu.async_copy(dst_hbm.at[pl.ds(ebase + m * HC, HC)],
                             rd.at[sl], isems[par])

        def _wait_idx(m, par):
            sl = _slot(m)
            pltpu.make_async_copy(src_hbm.at[pl.ds(ebase + m * HC, HC)],
                                  rs.at[sl], isems[par]).wait()
            pltpu.make_async_copy(dst_hbm.at[pl.ds(ebase + m * HC, HC)],
                                  rd.at[sl], isems[par]).wait()

        def _issue_g(m, b):
            la, ld, hf, gsem, _ = sets[b]
            sl = _slot(m)
            pltpu.async_copy(slog_hbm.at[rs.at[sl]], la, gsem)
            pltpu.async_copy(dlog_hbm.at[rd.at[sl]], ld, gsem)
            pltpu.async_copy(h_hbm.at[rs.at[sl]], hf, gsem)

        def _wait_g(m, b):
            la, ld, hf, gsem, _ = sets[b]
            sl = _slot(m)
            pltpu.make_async_copy(slog_hbm.at[rs.at[sl]], la, gsem).wait()
            pltpu.make_async_copy(dlog_hbm.at[rd.at[sl]], ld, gsem).wait()
            pltpu.make_async_copy(h_hbm.at[rs.at[sl]], hf, gsem).wait()

        def _issue_s(m, b):
            la, ld, hf, _, ssem = sets[b]
            sl = _slot(m)
            pltpu.async_copy(la, den_sp.at[rd.at[sl]], ssem, add=True)
            pltpu.async_copy(hf, out_sp.at[rd.at[sl]], ssem, add=True)

        def _wait_s(m, b):
            la, ld, hf, _, ssem = sets[b]
            sl = _slot(m)
            pltpu.make_async_copy(la, den_sp.at[rd.at[sl]], ssem).wait()
            pltpu.make_async_copy(hf, out_sp.at[rd.at[sl]], ssem).wait()

        _issue_idx(0, 0)
        _issue_idx(1, 1)
        _wait_idx(0, 0)
        _wait_idx(1, 1)
        _issue_g(0, 0)
        _issue_g(1, 1)
        _issue_idx(2, 0)
        _issue_idx(3, 1)

        def _quad(q, _):
            for bi in range(4):
                jh = 4 * q + bi
                la, ld, hf, _, _ = sets[bi]
                _wait_g(jh, bi)

                def _edge(i, _, la=la, ld=ld, hf=hf):
                    t = la[i, :] + ld[i, :]
                    ex = jnp.exp(jnp.maximum(t, 0.2 * t))
                    la[i, :] = ex
                    for hh in range(NH):
                        spl = jnp.full((HD,), ex[hoff + hh], _f32)
                        hf[i, pl.ds(HD * hh, HD)] = (
                            hf[i, pl.ds(HD * hh, HD)] * spl)
                    return 0

                lax.fori_loop(0, HC, _edge, 0, unroll=2)
                _issue_s(jh, bi)

                @pl.when(jh >= 2)
                def _():
                    _wait_s(jh - 2, (bi + 2) % 4)

                @pl.when(jh + 2 < NST)
                def _():
                    _wait_idx(jh + 2, bi % 2)
                    _issue_g(jh + 2, (bi + 2) % 4)

                @pl.when(jh + 4 < NST)
                def _():
                    _issue_idx(jh + 4, bi % 2)

            return 0

        lax.fori_loop(0, NST // 4, _quad, 0)
        _wait_s(NST - 2, 2)
        _wait_s(NST - 1, 3)
        plsc.subcore_barrier()

        def _fq(kk, _):
            r0 = rbase + RQ * kk
            pltpu.sync_copy(out_sp.at[pl.ds(r0, RQ)],
                            un_hbm.at[c, pl.ds(r0, RQ)])
            pltpu.sync_copy(den_sp.at[pl.ds(r0, RQ)],
                            den_hbm.at[c, pl.ds(r0, RQ)])
            return 0

        lax.fori_loop(0, nq, _fq, 0)

    return sc_edge


_sc_edge_w = _make_sc_edge(0)
_sc_edge_r = _make_sc_edge(NH)


def _make_norm_body(off):
    def _norm_body(un_ref, den_ref, o_ref):
        u = un_ref[0] + un_ref[1]
        d = den_ref[0] + den_ref[1]
        dh = d[:, off:off + NH]
        dr = jnp.broadcast_to(dh[:, :, None], dh.shape + (HD,))
        dr = dr.reshape(u.shape)
        o_ref[...] = jnp.maximum(u / (dr + 1e-16), 0.0)
    return _norm_body


def _tc_norm(un, den, off):
    blk = 1000
    return pl.pallas_call(
        _make_norm_body(off),
        grid=(N // blk,),
        in_specs=[
            pl.BlockSpec((NC, blk, CF), lambda i: (0, i, 0)),
            pl.BlockSpec((NC, blk, 2 * NH), lambda i: (0, i, 0)),
        ],
        out_specs=pl.BlockSpec((blk, CF), lambda i: (i, 0)),
        out_shape=jax.ShapeDtypeStruct((N, CF), _f32),
    )(un, den)


def _pad_edges(ei):
    src = jnp.pad(ei[0], (0, EP - E))
    dst = jnp.pad(ei[1], (0, EP - E), constant_values=N)
    return src, dst


def _layer(x_a, x_p, ew, er, pw_a, pb_a, pw_p, pb_p,
           as_w, ad_w, as_r, ad_r):
    wl_a = jnp.concatenate([_att_block(as_w), _att_block(ad_r)], axis=1)
    wl_p = jnp.concatenate([_att_block(ad_w), _att_block(as_r)], axis=1)
    h_a, la = _tc_proj(x_a, pw_a, pb_a, wl_a)
    h_p, lp = _tc_proj(x_p, pw_p, pb_p, wl_p)
    sw, dw = ew
    sr, dr = er
    un_p, den_p = _sc_edge_w(h_a, la, lp, sw, dw)
    un_a, den_a = _sc_edge_r(h_p, lp, la, sr, dr)
    out_a = _tc_norm(un_a, den_a, NH)
    out_p = _tc_norm(un_p, den_p, 0)
    return out_a, out_p


def kernel(x_author, x_paper, edge_index_writes, edge_index_rev_writes,
           p1w_author, p1b_author, p1w_paper, p1b_paper,
           a1s_writes, a1d_writes, a1s_rev, a1d_rev, k1w, k1b, q1,
           p2w_author, p2b_author, p2w_paper, p2b_paper,
           a2s_writes, a2d_writes, a2s_rev, a2d_rev, k2w, k2b, q2):
    ew = _pad_edges(edge_index_writes)
    er = _pad_edges(edge_index_rev_writes)
    a1, p1 = _layer(x_author, x_paper, ew, er,
                    p1w_author, p1b_author, p1w_paper, p1b_paper,
                    a1s_writes, a1d_writes, a1s_rev, a1d_rev)
    a2, p2 = _layer(a1, p1, ew, er,
                    p2w_author, p2b_author, p2w_paper, p2b_paper,
                    a2s_writes, a2d_writes, a2s_rev, a2d_rev)
    return jnp.stack([a2, p2])


# speedup vs baseline: 109.7132x; 1.0435x over previous
import functools

import jax
import jax.numpy as jnp
from jax import lax
from jax.experimental import pallas as pl
from jax.experimental.pallas import tpu as pltpu
from jax.experimental.pallas import tpu_sc as plsc

N = 10000
E = 320000
CF = 128
NH = 8
HD = 16
CW = CF + 2 * NH

NC = 2
NS = 16
NW = NC * NS
HC = 56
NST = 180
EPT = HC * NST
EP = EPT * NW
N1 = N + 8
ROWS_PT = 640
RQ = 40
RING = 8

_f32 = jnp.float32


def _proj_body(x_ref, w_ref, b_ref, wl_ref, cmb_ref, l_ref):
    h = jnp.dot(x_ref[...], w_ref[...], preferred_element_type=_f32)
    h = h + b_ref[...]
    l = jnp.dot(h, wl_ref[...], preferred_element_type=_f32)
    cmb_ref[:, :CF] = h
    cmb_ref[:, CF:] = l
    l_ref[...] = l


def _tc_proj(x, w, b, wl):
    blk = 1000
    return pl.pallas_call(
        _proj_body,
        grid=(N // blk,),
        in_specs=[
            pl.BlockSpec((blk, CF), lambda i: (i, 0)),
            pl.BlockSpec((CF, CF), lambda i: (0, 0)),
            pl.BlockSpec((1, CF), lambda i: (0, 0)),
            pl.BlockSpec((CF, 2 * NH), lambda i: (0, 0)),
        ],
        out_specs=[
            pl.BlockSpec((blk, CW), lambda i: (i, 0)),
            pl.BlockSpec((blk, 2 * NH), lambda i: (i, 0)),
        ],
        out_shape=[
            jax.ShapeDtypeStruct((N, CW), _f32),
            jax.ShapeDtypeStruct((N, 2 * NH), _f32),
        ],
    )(x, w.astype(_f32), b.reshape(1, CF).astype(_f32), wl)


def _att_block(att):
    eye = jnp.eye(NH, dtype=_f32)
    return (eye[:, None, :] * att[:, :, None]).reshape(CF, NH)


def _make_sc_edge(hoff):
    mesh = plsc.VectorSubcoreMesh(core_axis_name="c", subcore_axis_name="s")

    set_types = []
    for _ in range(4):
        set_types += [
            pltpu.VMEM((HC, CW), _f32),
            pltpu.VMEM((HC, 2 * NH), _f32),
            pltpu.SemaphoreType.DMA,
            pltpu.SemaphoreType.DMA,
        ]

    @functools.partial(
        pl.kernel,
        out_type=jax.ShapeDtypeStruct((NC, N, CW), _f32),
        mesh=mesh,
        compiler_params=pltpu.CompilerParams(use_tc_tiling_on_sc=False),
        scratch_types=[
            pltpu.VMEM((RING, HC), jnp.int32),
            pltpu.VMEM((RING, HC), jnp.int32),
            pltpu.VMEM_SHARED((N1, CW), _f32),
            pltpu.SemaphoreType.DMA,
            pltpu.SemaphoreType.DMA,
        ] + set_types,
    )
    def sc_edge(cmb_hbm, dlog_hbm, src_hbm, dst_hbm, acc_hbm,
                rs, rd, acc_sp, ie, io, *bufsets):
        sets = [bufsets[4 * b:4 * b + 4] for b in range(4)]
        isems = (ie, io)
        c = lax.axis_index("c")
        s = lax.axis_index("s")
        wid = s * NC + c
        ebase = wid * EPT

        cmb0 = sets[0][0]

        z16 = jnp.zeros((HD,), _f32)

        def _zrow(i, _):
            for kk in range(CW // HD):
                cmb0[i, pl.ds(HD * kk, HD)] = z16
            return 0

        lax.fori_loop(0, RQ, _zrow, 0)
        rbase = s * ROWS_PT
        nq = jnp.where(s == NS - 1, (N - (NS - 1) * ROWS_PT) // RQ,
                       ROWS_PT // RQ)

        def _zq(kk, _):
            pltpu.sync_copy(cmb0.at[pl.ds(0, RQ)],
                            acc_sp.at[pl.ds(rbase + RQ * kk, RQ)])
            return 0

        lax.fori_loop(0, nq, _zq, 0)
        plsc.subcore_barrier()

        def _slot(m):
            return lax.rem(m, RING)

        def _issue_idx(m, par):
            sl = _slot(m)
            pltpu.async_copy(src_hbm.at[pl.ds(ebase + m * HC, HC)],
                             rs.at[sl], isems[par])
            pltpu.async_copy(dst_hbm.at[pl.ds(ebase + m * HC, HC)],
                             rd.at[sl], isems[par])

        def _wait_idx(m, par):
            sl = _slot(m)
            pltpu.make_async_copy(src_hbm.at[pl.ds(ebase + m * HC, HC)],
                                  rs.at[sl], isems[par]).wait()
            pltpu.make_async_copy(dst_hbm.at[pl.ds(ebase + m * HC, HC)],
                                  rd.at[sl], isems[par]).wait()

        def _issue_g(m, b):
            cmb, ld, gsem, _ = sets[b]
            sl = _slot(m)
            pltpu.async_copy(cmb_hbm.at[rs.at[sl]], cmb, gsem)
            pltpu.async_copy(dlog_hbm.at[rd.at[sl]], ld, gsem)

        def _wait_g(m, b):
            cmb, ld, gsem, _ = sets[b]
            sl = _slot(m)
            pltpu.make_async_copy(cmb_hbm.at[rs.at[sl]], cmb, gsem).wait()
            pltpu.make_async_copy(dlog_hbm.at[rd.at[sl]], ld, gsem).wait()

        def _issue_s(m, b):
            cmb, ld, _, ssem = sets[b]
            sl = _slot(m)
            pltpu.async_copy(cmb, acc_sp.at[rd.at[sl]], ssem, add=True)

        def _wait_s(m, b):
            cmb, ld, _, ssem = sets[b]
            sl = _slot(m)
            pltpu.make_async_copy(cmb, acc_sp.at[rd.at[sl]], ssem).wait()

        _issue_idx(0, 0)
        _issue_idx(1, 1)
        _wait_idx(0, 0)
        _wait_idx(1, 1)
        _issue_g(0, 0)
        _issue_g(1, 1)
        _issue_idx(2, 0)
        _issue_idx(3, 1)

        def _quad(q, _):
            for bi in range(4):
                jh = 4 * q + bi
                cmb, ld, _, _ = sets[bi]
                _wait_g(jh, bi)

                def _edge(i, _, cmb=cmb, ld=ld):
                    t = cmb[i, pl.ds(CF, 2 * NH)] + ld[i, :]
                    ex = jnp.exp(jnp.maximum(t, 0.2 * t))
                    cmb[i, pl.ds(CF, 2 * NH)] = ex
                    for hh in range(NH):
                        spl = jnp.full((HD,), ex[hoff + hh], _f32)
                        cmb[i, pl.ds(HD * hh, HD)] = (
                            cmb[i, pl.ds(HD * hh, HD)] * spl)
                    return 0

                lax.fori_loop(0, HC, _edge, 0, unroll=2)
                _issue_s(jh, bi)

                @pl.when(jh >= 2)
                def _():
                    _wait_s(jh - 2, (bi + 2) % 4)

                @pl.when(jh + 2 < NST)
                def _():
                    _wait_idx(jh + 2, bi % 2)
                    _issue_g(jh + 2, (bi + 2) % 4)

                @pl.when(jh + 4 < NST)
                def _():
                    _issue_idx(jh + 4, bi % 2)

            return 0

        lax.fori_loop(0, NST // 4, _quad, 0)
        _wait_s(NST - 2, 2)
        _wait_s(NST - 1, 3)
        plsc.subcore_barrier()

        def _fq(kk, _):
            r0 = rbase + RQ * kk
            pltpu.sync_copy(acc_sp.at[pl.ds(r0, RQ)],
                            acc_hbm.at[c, pl.ds(r0, RQ)])
            return 0

        lax.fori_loop(0, nq, _fq, 0)

    return sc_edge


_sc_edge_w = _make_sc_edge(0)
_sc_edge_r = _make_sc_edge(NH)


def _norm_body(accp_ref, acca_ref, op_ref, oa_ref):
    for acc_ref, o_ref, off in ((accp_ref, op_ref, 0),
                                (acca_ref, oa_ref, NH)):
        u = acc_ref[0, :, :CF] + acc_ref[1, :, :CF]
        d = acc_ref[0, :, CF:] + acc_ref[1, :, CF:]
        dh = d[:, off:off + NH]
        dr = jnp.broadcast_to(dh[:, :, None], dh.shape + (HD,))
        dr = dr.reshape(u.shape)
        o_ref[...] = jnp.maximum(u / (dr + 1e-16), 0.0)


def _tc_norm(acc_p, acc_a):
    blk = 1000
    return pl.pallas_call(
        _norm_body,
        grid=(N // blk,),
        in_specs=[
            pl.BlockSpec((NC, blk, CW), lambda i: (0, i, 0)),
            pl.BlockSpec((NC, blk, CW), lambda i: (0, i, 0)),
        ],
        out_specs=[
            pl.BlockSpec((blk, CF), lambda i: (i, 0)),
            pl.BlockSpec((blk, CF), lambda i: (i, 0)),
        ],
        out_shape=[
            jax.ShapeDtypeStruct((N, CF), _f32),
            jax.ShapeDtypeStruct((N, CF), _f32),
        ],
    )(acc_p, acc_a)


def _pad_edges(ei):
    src = jnp.pad(ei[0], (0, EP - E))
    dst = jnp.pad(ei[1], (0, EP - E), constant_values=N)
    return src, dst


def _layer(x_a, x_p, ew, er, pw_a, pb_a, pw_p, pb_p,
           as_w, ad_w, as_r, ad_r):
    wl_a = jnp.concatenate([_att_block(as_w), _att_block(ad_r)], axis=1)
    wl_p = jnp.concatenate([_att_block(ad_w), _att_block(as_r)], axis=1)
    cmb_a, la = _tc_proj(x_a, pw_a, pb_a, wl_a)
    cmb_p, lp = _tc_proj(x_p, pw_p, pb_p, wl_p)
    la1 = jnp.pad(la, ((0, N1 - N), (0, 0)))
    lp1 = jnp.pad(lp, ((0, N1 - N), (0, 0)))
    sw, dw = ew
    sr, dr = er
    acc_p = _sc_edge_w(cmb_a, lp1, sw, dw)
    acc_a = _sc_edge_r(cmb_p, la1, sr, dr)
    return _tc_norm(acc_p, acc_a)


def kernel(x_author, x_paper, edge_index_writes, edge_index_rev_writes,
           p1w_author, p1b_author, p1w_paper, p1b_paper,
           a1s_writes, a1d_writes, a1s_rev, a1d_rev, k1w, k1b, q1,
           p2w_author, p2b_author, p2w_paper, p2b_paper,
           a2s_writes, a2d_writes, a2s_rev, a2d_rev, k2w, k2b, q2):
    ew = _pad_edges(edge_index_writes)
    er = _pad_edges(edge_index_rev_writes)
    p1, a1 = _layer(x_author, x_paper, ew, er,
                    p1w_author, p1b_author, p1w_paper, p1b_paper,
                    a1s_writes, a1d_writes, a1s_rev, a1d_rev)
    p2, a2 = _layer(a1, p1, ew, er,
                    p2w_author, p2b_author, p2w_paper, p2b_paper,
                    a2s_writes, a2d_writes, a2s_rev, a2d_rev)
    return jnp.stack([a2, p2])
